# baseline (speedup 1.0000x reference)
import functools

import jax
import jax.numpy as jnp
from jax.experimental import pallas as pl

_VOCABS = (6, 4, 4, 4, 6, 4, 4)
_OFF = (0, 6, 10, 14, 18, 24, 28)
_B = 16384
_BS = 4096

_DT0 = (((0,), (0,)), ((), ()))


def _body(uct_ref, unt_ref, t0, t1, t2, t3, t4, t5, t6, w1_ref, b1_ref,
          w2_ref, b2_ref, w3t_ref, b3_ref, out_ref):
    f32 = jnp.float32
    bf16 = jnp.bfloat16
    bs = out_ref.shape[1]

    s32 = jax.lax.broadcasted_iota(jnp.int32, (32, 8), 0)
    l32 = jax.lax.broadcasted_iota(jnp.int32, (32, 8), 1)
    fv = jnp.zeros((32, 8), jnp.int32)
    offv = jnp.zeros((32, 8), jnp.int32)
    for bnd, jump in zip(_OFF[1:], (6, 4, 4, 4, 6, 4)):
        step = (s32 >= bnd).astype(jnp.int32)
        fv = fv + step
        offv = offv + jump * step
    rt = (fv == l32).astype(f32)
    cmp_col = (s32 - offv).astype(f32)[:, 0:1]

    uctf = uct_ref[...].astype(f32)
    uct8 = jnp.concatenate([uctf, jnp.zeros((1, bs), f32)], axis=0)
    ucx_t = jnp.dot(rt, uct8, preferred_element_type=f32)
    mt = (ucx_t == cmp_col).astype(bf16)

    tabs = (t0, t1, t2, t3, t4, t5, t6)
    g = jnp.concatenate(
        [jnp.dot(t[...], w1_ref[8 * i:8 * i + 8, :],
                 preferred_element_type=f32) for i, t in enumerate(tabs)],
        axis=0)
    gs = jax.lax.broadcasted_iota(jnp.int32, (32, 128), 0)
    g = g + jnp.where(gs < 6, b1_ref[...].reshape(1, 128), 0.0)

    mt_ext = jnp.concatenate([mt, unt_ref[...].astype(bf16)], axis=0)
    g_ext = jnp.concatenate([g, w1_ref[56:58, :]], axis=0)

    b2col = jnp.transpose(b2_ref[...].reshape(1, 128))
    b3col = jnp.transpose(b3_ref[...].reshape(1, 64))

    h = jax.lax.dot_general(g_ext.astype(bf16), mt_ext, _DT0,
                            preferred_element_type=f32)
    h = jnp.maximum(h, 0.0)
    h = jax.lax.dot_general(w2_ref[...].astype(bf16), h.astype(bf16), _DT0,
                            preferred_element_type=f32) + b2col
    h = jnp.maximum(h, 0.0)
    out_ref[...] = (jnp.dot(w3t_ref[...].astype(bf16), h.astype(bf16),
                            preferred_element_type=f32) + b3col)


@functools.partial(jax.jit, static_argnames=("interpret",))
def kernel(u_cat, u_num, T_light, T_hum, T_care, T_size, T_climate, T_water,
           T_care_freq, W1, b1, W2, b2, W3, b3, interpret=False):
    tables = [T_light, T_hum, T_care, T_size, T_climate, T_water, T_care_freq]
    const = lambda s: pl.BlockSpec(s, lambda i: (0,) * len(s))
    grid = (_B // _BS,)
    out_t = pl.pallas_call(
        _body,
        grid=grid,
        in_specs=[
            pl.BlockSpec((7, _BS), lambda i: (0, i)),
            pl.BlockSpec((2, _BS), lambda i: (0, i)),
            *[const((v, 8)) for v in _VOCABS],
            const((58, 128)),
            const((128,)),
            const((128, 128)),
            const((128,)),
            const((64, 128)),
            const((64,)),
        ],
        out_specs=pl.BlockSpec((64, _BS), lambda i: (0, i)),
        out_shape=jax.ShapeDtypeStruct((64, _B), jnp.float32),
        interpret=interpret,
    )(u_cat.astype(jnp.int32).T, u_num.T, *tables, W1, b1, W2, b2, W3.T, b3)
    return out_t.T


# speedup vs baseline: 1.0864x; 1.0864x over previous
import functools

import jax
import jax.numpy as jnp
from jax.experimental import pallas as pl

_VOCABS = (6, 4, 4, 4, 6, 4, 4)
_OFF = (0, 6, 10, 14, 18, 24, 28)
_B = 16384
_BS = 8192

_DT0 = (((0,), (0,)), ((), ()))


def _body(uct_ref, unt_ref, t0, t1, t2, t3, t4, t5, t6, w1_ref, b1_ref,
          w2_ref, b2_ref, w3t_ref, b3_ref, out_ref):
    f32 = jnp.float32
    bf16 = jnp.bfloat16
    bs = out_ref.shape[1]

    s32 = jax.lax.broadcasted_iota(jnp.int32, (32, 8), 0)
    l32 = jax.lax.broadcasted_iota(jnp.int32, (32, 8), 1)
    fv = jnp.zeros((32, 8), jnp.int32)
    offv = jnp.zeros((32, 8), jnp.int32)
    for bnd, jump in zip(_OFF[1:], (6, 4, 4, 4, 6, 4)):
        step = (s32 >= bnd).astype(jnp.int32)
        fv = fv + step
        offv = offv + jump * step
    rt = (fv == l32).astype(f32)
    cmp_col = (s32 - offv).astype(f32)[:, 0:1]

    uctf = uct_ref[...].astype(f32)
    uct8 = jnp.concatenate([uctf, jnp.zeros((1, bs), f32)], axis=0)
    ucx_t = jnp.dot(rt, uct8, preferred_element_type=f32)
    mt = (ucx_t == cmp_col).astype(bf16)

    tabs = (t0, t1, t2, t3, t4, t5, t6)
    g = jnp.concatenate(
        [jnp.dot(t[...], w1_ref[8 * i:8 * i + 8, :],
                 preferred_element_type=f32) for i, t in enumerate(tabs)],
        axis=0)
    gs = jax.lax.broadcasted_iota(jnp.int32, (32, 128), 0)
    g = g + jnp.where(gs < 6, b1_ref[...].reshape(1, 128), 0.0)

    mt_ext = jnp.concatenate([mt, unt_ref[...].astype(bf16)], axis=0)
    g_ext = jnp.concatenate([g, w1_ref[56:58, :]], axis=0)

    b2col = jnp.transpose(b2_ref[...].reshape(1, 128))
    b3col = jnp.transpose(b3_ref[...].reshape(1, 64))

    h = jax.lax.dot_general(g_ext.astype(bf16), mt_ext, _DT0,
                            preferred_element_type=f32)
    h = jnp.maximum(h, 0.0)
    h = jax.lax.dot_general(w2_ref[...].astype(bf16), h.astype(bf16), _DT0,
                            preferred_element_type=f32) + b2col
    h = jnp.maximum(h, 0.0)
    out_ref[...] = (jnp.dot(w3t_ref[...].astype(bf16), h.astype(bf16),
                            preferred_element_type=f32) + b3col)


@functools.partial(jax.jit, static_argnames=("interpret",))
def kernel(u_cat, u_num, T_light, T_hum, T_care, T_size, T_climate, T_water,
           T_care_freq, W1, b1, W2, b2, W3, b3, interpret=False):
    tables = [T_light, T_hum, T_care, T_size, T_climate, T_water, T_care_freq]
    const = lambda s: pl.BlockSpec(s, lambda i: (0,) * len(s))
    grid = (_B // _BS,)
    out_t = pl.pallas_call(
        _body,
        grid=grid,
        in_specs=[
            pl.BlockSpec((7, _BS), lambda i: (0, i)),
            pl.BlockSpec((2, _BS), lambda i: (0, i)),
            *[const((v, 8)) for v in _VOCABS],
            const((58, 128)),
            const((128,)),
            const((128, 128)),
            const((128,)),
            const((64, 128)),
            const((64,)),
        ],
        out_specs=pl.BlockSpec((64, _BS), lambda i: (0, i)),
        out_shape=jax.ShapeDtypeStruct((64, _B), jnp.float32),
        interpret=interpret,
    )(u_cat.astype(jnp.int32).T, u_num.T, *tables, W1, b1, W2, b2, W3.T, b3)
    return out_t.T
